# baseline (speedup 1.0000x reference)
import functools

import jax
import jax.numpy as jnp
from jax import lax
from jax.experimental import pallas as pl
from jax.experimental.pallas import tpu as pltpu
from jax.experimental.pallas import tpu_sc as plsc

CUTOFF = 5.0
N_NODES = 10000
N_EDGES = 320000
HIDDEN = 128
N_RBF = 64

NC, NS = 2, 16
NW = NC * NS
K = 80
CHUNKS = 16
PHASES = 8
E_HALF = NW * K * CHUNKS
E_PAD = PHASES * E_HALF
N_PAD = 10240
ROWS_PER_TILE = N_PAD // NS


def _filter_body(d_ref, c_ref, g_ref, w1_ref, b1_ref, w2_ref, b2_ref, o_ref):
    d = d_ref[...]
    g = g_ref[0, 0]
    diff = d - c_ref[...]
    rbf = jnp.exp(-g * diff * diff)
    h = jnp.dot(rbf, w1_ref[...], preferred_element_type=jnp.float32) + b1_ref[...]
    h = h * jax.nn.sigmoid(h)
    w = jnp.dot(h, w2_ref[...], preferred_element_type=jnp.float32) + b2_ref[...]
    xc = jnp.clip(d * (1.0 / CUTOFF), 0.0, 1.0)
    cc = 0.5 * (jnp.cos(jnp.pi * xc) + 1.0) * (xc < 1.0).astype(jnp.float32)
    o_ref[...] = w * cc


def _filter_call(dist_half, centers, gamma, W1, b1, W2, b2):
    BE = 2048
    return pl.pallas_call(
        _filter_body,
        grid=(E_HALF // BE,),
        in_specs=[
            pl.BlockSpec((BE, 1), lambda i: (i, 0)),
            pl.BlockSpec((1, N_RBF), lambda i: (0, 0)),
            pl.BlockSpec(memory_space=pltpu.SMEM),
            pl.BlockSpec((N_RBF, HIDDEN), lambda i: (0, 0)),
            pl.BlockSpec((1, HIDDEN), lambda i: (0, 0)),
            pl.BlockSpec((HIDDEN, HIDDEN), lambda i: (0, 0)),
            pl.BlockSpec((1, HIDDEN), lambda i: (0, 0)),
        ],
        out_specs=pl.BlockSpec((BE, HIDDEN), lambda i: (i, 0)),
        out_shape=jax.ShapeDtypeStruct((E_HALF, HIDDEN), jnp.float32),
    )(
        dist_half.reshape(E_HALF, 1),
        centers.reshape(1, N_RBF),
        gamma.reshape(1, 1),
        W1,
        b1.reshape(1, HIDDEN),
        W2,
        b2.reshape(1, HIDDEN),
    )


def _xd_body(x_ref, wd_ref, o_ref):
    o_ref[...] = jnp.dot(x_ref[...], wd_ref[...],
                         preferred_element_type=jnp.float32)


def _xd_call(x, Wd):
    BN = 2000
    return pl.pallas_call(
        _xd_body,
        grid=(N_NODES // BN,),
        in_specs=[
            pl.BlockSpec((BN, HIDDEN), lambda i: (i, 0)),
            pl.BlockSpec((HIDDEN, HIDDEN), lambda i: (0, 0)),
        ],
        out_specs=pl.BlockSpec((BN, HIDDEN), lambda i: (i, 0)),
        out_shape=jax.ShapeDtypeStruct((N_NODES, HIDDEN), jnp.float32),
    )(x, Wd)


def _combine_body(*refs):
    o_ref = refs[-1]
    total = refs[0][...]
    for r in refs[1:-1]:
        total = total + r[...]
    o_ref[...] = total


def _combine_call(*parts):
    BN = 2000
    spec = pl.BlockSpec((BN, HIDDEN), lambda i: (i, 0))
    return pl.pallas_call(
        _combine_body,
        grid=(N_NODES // BN,),
        in_specs=[spec] * len(parts),
        out_specs=spec,
        out_shape=jax.ShapeDtypeStruct((N_NODES, HIDDEN), jnp.float32),
    )(*parts)


def _sc_body(xd_h, wc_h, src_h, dst_h, out_h,
             idx_s0, idx_d0, rows0, wcb0,
             idx_s1, idx_d1, rows1, wcb1,
             acc, sem0, sem1):
    c = lax.axis_index("c")
    s = lax.axis_index("s")
    wid = c * NS + s
    base_w = wid * CHUNKS * K

    @plsc.parallel_loop(0, K)
    def _zrow(i):
        for j in range(HIDDEN // 16):
            rows0[i, pl.ds(j * 16, 16)] = jnp.zeros((16,), jnp.float32)

    zbase = s * ROWS_PER_TILE
    n_full = ROWS_PER_TILE // K
    for t in range(n_full):
        pltpu.sync_copy(rows0, acc.at[pl.ds(zbase + t * K, K)])
    plsc.subcore_barrier()

    def _start(ci, idx_s, idx_d, rows, wcb, sem):
        base = base_w + ci * K
        pltpu.sync_copy(src_h.at[pl.ds(base, K)], idx_s)
        pltpu.sync_copy(dst_h.at[pl.ds(base, K)], idx_d)
        pltpu.async_copy(xd_h.at[idx_s], rows, sem)
        pltpu.async_copy(wc_h.at[pl.ds(base, K)], wcb, sem)

    def _finish(idx_s, idx_d, rows, wcb, sem):
        pltpu.make_async_copy(xd_h.at[idx_s], rows, sem).wait()
        pltpu.make_async_copy(wc_h.at[pl.ds(0, K)], wcb, sem).wait()

        @plsc.parallel_loop(0, K, unroll=2)
        def _mulrow(i):
            for j in range(HIDDEN // 16):
                sl = pl.ds(j * 16, 16)
                rows[i, sl] = rows[i, sl] * wcb[i, sl]

        pltpu.sync_copy(rows, acc.at[idx_d], add=True)

    buf0 = (idx_s0, idx_d0, rows0, wcb0, sem0)
    buf1 = (idx_s1, idx_d1, rows1, wcb1, sem1)
    _start(0, *buf0)

    def _pair(j, carry):
        _start(2 * j + 1, *buf1)
        _finish(*buf0)

        @pl.when(j < CHUNKS // 2 - 1)
        def _():
            _start(2 * j + 2, *buf0)
        _finish(*buf1)
        return carry
    lax.fori_loop(0, CHUNKS // 2, _pair, 0)
    plsc.subcore_barrier()

    for t in range(n_full):
        pltpu.sync_copy(acc.at[pl.ds(zbase + t * K, K)], rows0)
        pltpu.sync_copy(rows0, out_h.at[c, pl.ds(zbase + t * K, K)])


_sc_call = functools.partial(
    pl.kernel,
    out_type=jax.ShapeDtypeStruct((NC, N_PAD, HIDDEN), jnp.float32),
    mesh=plsc.VectorSubcoreMesh(core_axis_name="c", subcore_axis_name="s"),
    scratch_types=[
        pltpu.VMEM((K,), jnp.int32),
        pltpu.VMEM((K,), jnp.int32),
        pltpu.VMEM((K, HIDDEN), jnp.float32),
        pltpu.VMEM((K, HIDDEN), jnp.float32),
        pltpu.VMEM((K,), jnp.int32),
        pltpu.VMEM((K,), jnp.int32),
        pltpu.VMEM((K, HIDDEN), jnp.float32),
        pltpu.VMEM((K, HIDDEN), jnp.float32),
        pltpu.VMEM_SHARED((N_PAD, HIDDEN), jnp.float32),
        pltpu.SemaphoreType.DMA,
        pltpu.SemaphoreType.DMA,
    ],
)(_sc_body)


def kernel(x, edge_index, distances, centers, gamma, W1, b1, W2, b2, Wd):
    src = edge_index[0].astype(jnp.int32)
    dst = edge_index[1].astype(jnp.int32)
    pad = E_PAD - N_EDGES
    dist_pad = jnp.concatenate(
        [distances, jnp.full((pad,), CUTOFF, jnp.float32)])
    src_p = jnp.concatenate([src, jnp.zeros((pad,), jnp.int32)])
    dst_p = jnp.concatenate([dst, jnp.zeros((pad,), jnp.int32)])

    xd = _xd_call(x, Wd)
    centers32 = centers.astype(jnp.float32)
    gamma32 = gamma.astype(jnp.float32)
    partials = []
    for ph in range(PHASES):
        sl = slice(ph * E_HALF, (ph + 1) * E_HALF)
        wc = _filter_call(dist_pad[sl], centers32, gamma32, W1, b1, W2, b2)
        parts = _sc_call(xd, wc, src_p[sl], dst_p[sl])
        partials.extend([parts[0], parts[1]])
    return _combine_call(*partials)


# speedup vs baseline: 1.1176x; 1.1176x over previous
import functools

import jax
import jax.numpy as jnp
from jax import lax
from jax.experimental import pallas as pl
from jax.experimental.pallas import tpu as pltpu
from jax.experimental.pallas import tpu_sc as plsc

CUTOFF = 5.0
N_NODES = 10000
N_EDGES = 320000
HIDDEN = 128
N_RBF = 64

NC, NS = 2, 16
NW = NC * NS
K = 80
CHUNKS = 32
PHASES = 4
E_HALF = NW * K * CHUNKS
E_PAD = PHASES * E_HALF
N_PAD = 10240
ROWS_PER_TILE = N_PAD // NS


def _filter_body(d_ref, c_ref, g_ref, w1_ref, b1_ref, w2_ref, b2_ref, o_ref):
    d = d_ref[...]
    g = g_ref[0, 0]
    diff = d - c_ref[...]
    rbf = jnp.exp(-g * diff * diff)
    h = jnp.dot(rbf, w1_ref[...], preferred_element_type=jnp.float32) + b1_ref[...]
    h = h * jax.nn.sigmoid(h)
    w = jnp.dot(h, w2_ref[...], preferred_element_type=jnp.float32) + b2_ref[...]
    xc = jnp.clip(d * (1.0 / CUTOFF), 0.0, 1.0)
    cc = 0.5 * (jnp.cos(jnp.pi * xc) + 1.0) * (xc < 1.0).astype(jnp.float32)
    o_ref[...] = w * cc


def _filter_call(dist_half, centers, gamma, W1, b1, W2, b2):
    BE = 2048
    return pl.pallas_call(
        _filter_body,
        grid=(E_HALF // BE,),
        in_specs=[
            pl.BlockSpec((BE, 1), lambda i: (i, 0)),
            pl.BlockSpec((1, N_RBF), lambda i: (0, 0)),
            pl.BlockSpec(memory_space=pltpu.SMEM),
            pl.BlockSpec((N_RBF, HIDDEN), lambda i: (0, 0)),
            pl.BlockSpec((1, HIDDEN), lambda i: (0, 0)),
            pl.BlockSpec((HIDDEN, HIDDEN), lambda i: (0, 0)),
            pl.BlockSpec((1, HIDDEN), lambda i: (0, 0)),
        ],
        out_specs=pl.BlockSpec((BE, HIDDEN), lambda i: (i, 0)),
        out_shape=jax.ShapeDtypeStruct((E_HALF, HIDDEN), jnp.float32),
    )(
        dist_half.reshape(E_HALF, 1),
        centers.reshape(1, N_RBF),
        gamma.reshape(1, 1),
        W1,
        b1.reshape(1, HIDDEN),
        W2,
        b2.reshape(1, HIDDEN),
    )


def _xd_body(x_ref, wd_ref, o_ref):
    o_ref[...] = jnp.dot(x_ref[...], wd_ref[...],
                         preferred_element_type=jnp.float32)


def _xd_call(x, Wd):
    BN = 2000
    return pl.pallas_call(
        _xd_body,
        grid=(N_NODES // BN,),
        in_specs=[
            pl.BlockSpec((BN, HIDDEN), lambda i: (i, 0)),
            pl.BlockSpec((HIDDEN, HIDDEN), lambda i: (0, 0)),
        ],
        out_specs=pl.BlockSpec((BN, HIDDEN), lambda i: (i, 0)),
        out_shape=jax.ShapeDtypeStruct((N_NODES, HIDDEN), jnp.float32),
    )(x, Wd)


def _combine_body(*refs):
    o_ref = refs[-1]
    total = refs[0][...]
    for r in refs[1:-1]:
        total = total + r[...]
    o_ref[...] = total


def _combine_call(*parts):
    BN = 2000
    spec = pl.BlockSpec((BN, HIDDEN), lambda i: (i, 0))
    return pl.pallas_call(
        _combine_body,
        grid=(N_NODES // BN,),
        in_specs=[spec] * len(parts),
        out_specs=spec,
        out_shape=jax.ShapeDtypeStruct((N_NODES, HIDDEN), jnp.float32),
    )(*parts)


def _sc_body(xd_h, wc_h, src_h, dst_h, out_h,
             idx_s0, idx_d0, rows0, wcb0,
             idx_s1, idx_d1, rows1, wcb1,
             acc, sem0, sem1):
    c = lax.axis_index("c")
    s = lax.axis_index("s")
    wid = c * NS + s
    base_w = wid * CHUNKS * K

    @plsc.parallel_loop(0, K)
    def _zrow(i):
        for j in range(HIDDEN // 16):
            rows0[i, pl.ds(j * 16, 16)] = jnp.zeros((16,), jnp.float32)

    zbase = s * ROWS_PER_TILE
    n_full = ROWS_PER_TILE // K
    for t in range(n_full):
        pltpu.sync_copy(rows0, acc.at[pl.ds(zbase + t * K, K)])
    plsc.subcore_barrier()

    def _start(ci, idx_s, idx_d, rows, wcb, sem):
        base = base_w + ci * K
        pltpu.sync_copy(src_h.at[pl.ds(base, K)], idx_s)
        pltpu.sync_copy(dst_h.at[pl.ds(base, K)], idx_d)
        pltpu.async_copy(xd_h.at[idx_s], rows, sem)
        pltpu.async_copy(wc_h.at[pl.ds(base, K)], wcb, sem)

    def _finish(idx_s, idx_d, rows, wcb, sem):
        pltpu.make_async_copy(xd_h.at[idx_s], rows, sem).wait()
        pltpu.make_async_copy(wc_h.at[pl.ds(0, K)], wcb, sem).wait()

        @plsc.parallel_loop(0, K, unroll=2)
        def _mulrow(i):
            for j in range(HIDDEN // 16):
                sl = pl.ds(j * 16, 16)
                rows[i, sl] = rows[i, sl] * wcb[i, sl]

        pltpu.sync_copy(rows, acc.at[idx_d], add=True)

    buf0 = (idx_s0, idx_d0, rows0, wcb0, sem0)
    buf1 = (idx_s1, idx_d1, rows1, wcb1, sem1)
    _start(0, *buf0)

    def _pair(j, carry):
        _start(2 * j + 1, *buf1)
        _finish(*buf0)

        @pl.when(j < CHUNKS // 2 - 1)
        def _():
            _start(2 * j + 2, *buf0)
        _finish(*buf1)
        return carry
    lax.fori_loop(0, CHUNKS // 2, _pair, 0)
    plsc.subcore_barrier()

    for t in range(n_full):
        pltpu.sync_copy(acc.at[pl.ds(zbase + t * K, K)], rows0)
        pltpu.sync_copy(rows0, out_h.at[c, pl.ds(zbase + t * K, K)])


_sc_call = functools.partial(
    pl.kernel,
    out_type=jax.ShapeDtypeStruct((NC, N_PAD, HIDDEN), jnp.float32),
    mesh=plsc.VectorSubcoreMesh(core_axis_name="c", subcore_axis_name="s"),
    scratch_types=[
        pltpu.VMEM((K,), jnp.int32),
        pltpu.VMEM((K,), jnp.int32),
        pltpu.VMEM((K, HIDDEN), jnp.float32),
        pltpu.VMEM((K, HIDDEN), jnp.float32),
        pltpu.VMEM((K,), jnp.int32),
        pltpu.VMEM((K,), jnp.int32),
        pltpu.VMEM((K, HIDDEN), jnp.float32),
        pltpu.VMEM((K, HIDDEN), jnp.float32),
        pltpu.VMEM_SHARED((N_PAD, HIDDEN), jnp.float32),
        pltpu.SemaphoreType.DMA,
        pltpu.SemaphoreType.DMA,
    ],
)(_sc_body)


def kernel(x, edge_index, distances, centers, gamma, W1, b1, W2, b2, Wd):
    src = edge_index[0].astype(jnp.int32)
    dst = edge_index[1].astype(jnp.int32)
    pad = E_PAD - N_EDGES
    dist_pad = jnp.concatenate(
        [distances, jnp.full((pad,), CUTOFF, jnp.float32)])
    src_p = jnp.concatenate([src, jnp.zeros((pad,), jnp.int32)])
    dst_p = jnp.concatenate([dst, jnp.zeros((pad,), jnp.int32)])

    xd = _xd_call(x, Wd)
    centers32 = centers.astype(jnp.float32)
    gamma32 = gamma.astype(jnp.float32)
    partials = []
    for ph in range(PHASES):
        sl = slice(ph * E_HALF, (ph + 1) * E_HALF)
        wc = _filter_call(dist_pad[sl], centers32, gamma32, W1, b1, W2, b2)
        parts = _sc_call(xd, wc, src_p[sl], dst_p[sl])
        partials.extend([parts[0], parts[1]])
    return _combine_call(*partials)
